# baseline (speedup 1.0000x reference)
import functools

import jax
import jax.numpy as jnp
from jax import lax
from jax.experimental import pallas as pl
from jax.experimental.pallas import tpu as pltpu
from jax.experimental.pallas import tpu_sc as plsc

N = 10000
E0 = 320000
E = E0 + N
D_IN = 128
DH = 64
SUB = 10

NC = 2
NS = 16
NW = NC * NS
R = 81
ROWS = NW * R
E_PAD = ROWS * 128
NPAD = 10240
NSL = NPAD // NS

_f32 = jnp.float32
_i32 = jnp.int32


def _attn_scalars(h, asrc2d, adst2d, ss_ref, sd_ref, c_ref):
    ss = jnp.dot(h, asrc2d, preferred_element_type=_f32)
    sd = jnp.dot(h, adst2d, preferred_element_type=_f32)
    ss_ref[...] = ss
    sd_ref[...] = sd
    craw = jnp.max(ss) + jnp.max(sd)
    c = jnp.where(craw > 0.0, craw, 0.2 * craw)
    c_ref[...] = jnp.full((16,), c, _f32)


def _tc_head_body(x_ref, w_ref, asrc_ref, adst_ref, h_ref, ss_ref, sd_ref, c_ref):
    h = jnp.dot(x_ref[...], w_ref[...], preferred_element_type=_f32)
    h_ref[...] = h
    _attn_scalars(h, asrc_ref[...], adst_ref[...], ss_ref, sd_ref, c_ref)


def _tc_head(x, w, asrc, adst):
    return pl.pallas_call(
        _tc_head_body,
        out_shape=[
            jax.ShapeDtypeStruct((N, DH), _f32),
            jax.ShapeDtypeStruct((N, 1), _f32),
            jax.ShapeDtypeStruct((N, 1), _f32),
            jax.ShapeDtypeStruct((16,), _f32),
        ],
    )(x, w, asrc, adst)


def _tc_mid_body(p0_ref, p1_ref, d0_ref, d1_ref, b_ref, w_ref, asrc_ref,
                 adst_ref, h_ref, ss_ref, sd_ref, c_ref):
    den = d0_ref[...] + d1_ref[...] + 1e-16
    hin = jnp.maximum((p0_ref[...] + p1_ref[...]) / den + b_ref[...][None, :], 0.0)
    h = jnp.dot(hin, w_ref[...], preferred_element_type=_f32)
    h_ref[...] = h
    _attn_scalars(h, asrc_ref[...], adst_ref[...], ss_ref, sd_ref, c_ref)


def _tc_mid(p0, p1, d0, d1, b, w, asrc, adst):
    return pl.pallas_call(
        _tc_mid_body,
        out_shape=[
            jax.ShapeDtypeStruct((N, DH), _f32),
            jax.ShapeDtypeStruct((N, 1), _f32),
            jax.ShapeDtypeStruct((N, 1), _f32),
            jax.ShapeDtypeStruct((16,), _f32),
        ],
    )(p0, p1, d0, d1, b, w, asrc, adst)


def _tc_rowsum_body(p0_ref, p1_ref, d0_ref, d1_ref, b_ref, wrep_ref, rs_ref):
    den = d0_ref[...] + d1_ref[...] + 1e-16
    h = jnp.maximum((p0_ref[...] + p1_ref[...]) / den + b_ref[...][None, :], 0.0)
    ones = jnp.ones((DH, 1), _f32)
    rs_ref[...] = jnp.dot(h * wrep_ref[...], ones, preferred_element_type=_f32)


def _tc_rowsum(p0, p1, d0, d1, b, wrep):
    return pl.pallas_call(
        _tc_rowsum_body,
        out_shape=jax.ShapeDtypeStruct((N, 1), _f32),
    )(p0, p1, d0, d1, b, wrep)


def _tc_fold_body(p_ref, bout_ref, o_ref):
    o_ref[...] = jnp.sum(p_ref[...], axis=1, keepdims=True) + bout_ref[...][None, :]


def _tc_fold(p, bout):
    return pl.pallas_call(
        _tc_fold_body,
        out_shape=jax.ShapeDtypeStruct((N // SUB, 1), _f32),
    )(p, bout)


_mesh = plsc.VectorSubcoreMesh(core_axis_name="c", subcore_axis_name="s")
_sc_params = pltpu.CompilerParams(use_tc_tiling_on_sc=False,
                                  needs_layout_passes=False)

@functools.partial(
    pl.kernel,
    out_type=[
        jax.ShapeDtypeStruct((NPAD,), _f32),
        jax.ShapeDtypeStruct((NPAD,), _f32),
        jax.ShapeDtypeStruct((NPAD, DH), _f32),
        jax.ShapeDtypeStruct((NPAD, DH), _f32),
    ],
    mesh=_mesh,
    compiler_params=_sc_params,
    scratch_types=[
        pltpu.VMEM((R, 128), _i32),
        pltpu.VMEM((R, 128), _i32),
        pltpu.VMEM((R, 128), _f32),
        pltpu.VMEM((N,), _f32),
        pltpu.VMEM((N,), _f32),
        pltpu.VMEM((16,), _f32),
        pltpu.VMEM((64, DH), _f32),
        pltpu.VMEM((NSL,), _f32),
        pltpu.VMEM((128, DH), _f32),
        pltpu.VMEM((128, DH), _f32),
        pltpu.VMEM((128, DH), _f32),
        pltpu.VMEM((128, DH), _f32),
        pltpu.VMEM_SHARED((NPAD,), _f32),
        pltpu.VMEM_SHARED((NPAD, DH), _f32),
        pltpu.SemaphoreType.DMA,
        pltpu.SemaphoreType.DMA,
        pltpu.SemaphoreType.DMA,
        pltpu.SemaphoreType.DMA,
        pltpu.SemaphoreType.DMA,
        pltpu.SemaphoreType.DMA,
    ],
)
def _sc_gat_layer(src_hbm, dst_hbm, ss_hbm, sd_hbm, cv_hbm, h_hbm,
                  d0_hbm, d1_hbm, o0_hbm, o1_hbm,
                  vsrc, vdst, vp, ssv, sdv, cbuf, zbuf, zdbuf,
                  rin0, rin1, rout0, rout1, dsh, osh,
                  semz, semd, semg0, semg1, sems0, sems1):
    c = lax.axis_index("c")
    s = lax.axis_index("s")
    wid = c * NS + s
    base = wid * R
    row0 = s * NSL

    pltpu.async_copy(src_hbm.at[pl.ds(base, R)], vsrc, semz)
    pltpu.async_copy(dst_hbm.at[pl.ds(base, R)], vdst, semz)
    pltpu.async_copy(ss_hbm, ssv, semz)
    pltpu.async_copy(sd_hbm, sdv, semz)
    pltpu.sync_copy(cv_hbm, cbuf)

    for r in range(64):
        for t in range(DH // 16):
            zbuf[r, pl.ds(t * 16, 16)] = jnp.zeros((16,), _f32)
    for t in range(NSL // 16):
        zdbuf[pl.ds(t * 16, 16)] = jnp.zeros((16,), _f32)
    for t in range(NSL // 64):
        pltpu.async_copy(zbuf, osh.at[pl.ds(row0 + t * 64, 64)], semg0)
    pltpu.async_copy(zdbuf, dsh.at[pl.ds(s * NSL, NSL)], semg1)

    pltpu.make_async_copy(src_hbm.at[pl.ds(base, R)], vsrc, semz).wait()
    pltpu.make_async_copy(src_hbm.at[pl.ds(base, R)], vdst, semz).wait()
    pltpu.make_async_copy(ss_hbm, ssv, semz).wait()
    pltpu.make_async_copy(sd_hbm, sdv, semz).wait()
    cv = cbuf[...]

    def pbody(j, carry):
        for k in range(8):
            sl = pl.ds(k * 16, 16)
            e = (plsc.load_gather(ssv, [vsrc[j, sl]])
                 + plsc.load_gather(sdv, [vdst[j, sl]]))
            e = jnp.where(e > 0.0, e, 0.2 * e)
            gsl = jnp.exp(e - cv)
            vp[j, sl] = gsl
        return carry

    lax.fori_loop(0, R, pbody, 0)

    @pl.when(wid == NW - 1)
    def _():
        zero = jnp.zeros((16,), _f32)
        pad0 = E - (NW - 1) * R * 128
        jpad, kpad = pad0 // 128, pad0 % 128
        for k in range(kpad // 16, 8):
            vp[jpad, pl.ds(k * 16, 16)] = zero

        def zrow(j, carry):
            for k in range(8):
                vp[j, pl.ds(k * 16, 16)] = zero
            return carry

        lax.fori_loop(jpad + 1, R, zrow, 0)

    for t in range(NSL // 64):
        pltpu.make_async_copy(o0_hbm.at[pl.ds(0, 64)], zbuf, semg0).wait()
    pltpu.make_async_copy(d0_hbm.at[pl.ds(0, NSL)], zdbuf, semg1).wait()
    plsc.subcore_barrier()

    def dfire(j, carry):
        pltpu.async_copy(vp.at[j], dsh.at[vdst.at[j]], semd, add=True)
        return carry

    lax.fori_loop(0, R, dfire, 0)

    def scale(j, rin, rout):
        for g in range(8):
            av = vp[j, pl.ds(g * 16, 16)]
            for u in range(16):
                i = g * 16 + u
                a = av[u]
                for t in range(DH // 16):
                    sl = pl.ds(t * 16, 16)
                    rout[i, sl] = rin[i, sl] * a

    def fire_gather(j, rin, semg):
        pltpu.async_copy(h_hbm.at[vsrc.at[j]], rin, semg)

    def fire_scatter(j, rout, sems):
        pltpu.async_copy(rout, osh.at[vdst.at[j]], sems, add=True)

    def drain(buf, sem):
        pltpu.make_async_copy(h_hbm.at[pl.ds(0, 128)], buf, sem).wait()

    fire_gather(0, rin0, semg0)
    fire_gather(1, rin1, semg1)

    drain(rin0, semg0)
    scale(0, rin0, rout0)
    fire_scatter(0, rout0, sems0)
    fire_gather(2, rin0, semg0)
    drain(rin1, semg1)
    scale(1, rin1, rout1)
    fire_scatter(1, rout1, sems1)
    fire_gather(3, rin1, semg1)

    def body(g, carry):
        j0 = 2 * g
        drain(rin0, semg0)
        drain(rout0, sems0)
        scale(j0, rin0, rout0)
        fire_scatter(j0, rout0, sems0)
        fire_gather(j0 + 2, rin0, semg0)
        drain(rin1, semg1)
        drain(rout1, sems1)
        scale(j0 + 1, rin1, rout1)
        fire_scatter(j0 + 1, rout1, sems1)

        @pl.when(j0 + 3 < R)
        def _():
            fire_gather(j0 + 3, rin1, semg1)

        return carry

    lax.fori_loop(1, (R - 1) // 2, body, 0)

    drain(rin0, semg0)
    drain(rout0, sems0)
    scale(R - 1, rin0, rout0)
    fire_scatter(R - 1, rout0, sems0)
    drain(rout0, sems0)
    drain(rout1, sems1)

    def ddrain(j, carry):
        pltpu.make_async_copy(d0_hbm.at[pl.ds(0, 128)], vp.at[j], semd).wait()
        return carry

    lax.fori_loop(0, R, ddrain, 0)

    plsc.subcore_barrier()

    sl1 = pl.ds(s * NSL, NSL)
    slr = pl.ds(row0, NSL)

    @pl.when(c == 0)
    def _():
        pltpu.sync_copy(dsh.at[sl1], d0_hbm.at[sl1])
        pltpu.sync_copy(osh.at[slr], o0_hbm.at[slr])

    @pl.when(c == 1)
    def _():
        pltpu.sync_copy(dsh.at[sl1], d1_hbm.at[sl1])
        pltpu.sync_copy(osh.at[slr], o1_hbm.at[slr])


def kernel(x, edge_index, W1, a_src1, a_dst1, b1, W2, a_src2, a_dst2, b2,
           Wout, bout):
    loop = jnp.arange(N, dtype=_i32)
    padi = jnp.zeros((E_PAD - E,), _i32)
    src = jnp.concatenate([edge_index[0].astype(_i32), loop, padi]).reshape(ROWS, 128)
    dst = jnp.concatenate([edge_index[1].astype(_i32), loop, padi]).reshape(ROWS, 128)

    asrc1r = jnp.reshape(a_src1, (DH, 1))
    adst1r = jnp.reshape(a_dst1, (DH, 1))
    asrc2r = jnp.reshape(a_src2, (DH, 1))
    adst2r = jnp.reshape(a_dst2, (DH, 1))

    h1, ss1, sd1, c1 = _tc_head(x, W1, asrc1r, adst1r)
    d0, d1, o0, o1 = _sc_gat_layer(src, dst, jnp.reshape(ss1, (N,)),
                                   jnp.reshape(sd1, (N,)), c1, h1)

    d0r = jnp.reshape(d0[:N], (N, 1))
    d1r = jnp.reshape(d1[:N], (N, 1))
    h2, ss2, sd2, c2 = _tc_mid(o0[:N], o1[:N], d0r, d1r, b1, W2, asrc2r, adst2r)
    e0, e1, q0, q1 = _sc_gat_layer(src, dst, jnp.reshape(ss2, (N,)),
                                   jnp.reshape(sd2, (N,)), c2, h2)

    e0r = jnp.reshape(e0[:N], (N, 1))
    e1r = jnp.reshape(e1[:N], (N, 1))
    wrep = jnp.tile(jnp.reshape(Wout[:, 0], (SUB, DH)), (N // SUB, 1))
    rs = _tc_rowsum(q0[:N], q1[:N], e0r, e1r, b2, wrep)
    out = _tc_fold(jnp.reshape(rs, (N // SUB, SUB)), bout)
    return out


# speedup vs baseline: 1.2040x; 1.2040x over previous
import functools

import jax
import jax.numpy as jnp
from jax import lax
from jax.experimental import pallas as pl
from jax.experimental.pallas import tpu as pltpu
from jax.experimental.pallas import tpu_sc as plsc

N = 10000
E0 = 320000
D_IN = 128
DH = 64
SUB = 10

NC = 2
NS = 16
NW = NC * NS
R = 79
ROWS = E0 // 128
NPAD = 10240
NSL = NPAD // NS

_f32 = jnp.float32
_i32 = jnp.int32


def _attn_scalars(h, asrc2d, adst2d, ss_ref, sd_ref, c_ref, pself_ref):
    ss = jnp.dot(h, asrc2d, preferred_element_type=_f32)
    sd = jnp.dot(h, adst2d, preferred_element_type=_f32)
    ss_ref[...] = ss
    sd_ref[...] = sd
    craw = jnp.max(ss) + jnp.max(sd)
    c = jnp.where(craw > 0.0, craw, 0.2 * craw)
    c_ref[...] = jnp.full((16,), c, _f32)
    eself = ss + sd
    eself = jnp.where(eself > 0.0, eself, 0.2 * eself)
    pself_ref[...] = jnp.exp(eself - c)


_scalar_out = [
    jax.ShapeDtypeStruct((N, DH), _f32),
    jax.ShapeDtypeStruct((N, 1), _f32),
    jax.ShapeDtypeStruct((N, 1), _f32),
    jax.ShapeDtypeStruct((16,), _f32),
    jax.ShapeDtypeStruct((N, 1), _f32),
]


def _tc_head_body(x_ref, w_ref, asrc_ref, adst_ref,
                  h_ref, ss_ref, sd_ref, c_ref, pself_ref):
    h = jnp.dot(x_ref[...], w_ref[...], preferred_element_type=_f32)
    h_ref[...] = h
    _attn_scalars(h, asrc_ref[...], adst_ref[...], ss_ref, sd_ref, c_ref,
                  pself_ref)


def _tc_head(x, w, asrc, adst):
    return pl.pallas_call(_tc_head_body, out_shape=_scalar_out)(x, w, asrc, adst)


def _tc_mid_body(p0_ref, p1_ref, d0_ref, d1_ref, ps_ref, ht_ref, b_ref,
                 w_ref, asrc_ref, adst_ref,
                 h_ref, ss_ref, sd_ref, c_ref, pself_ref):
    ps = ps_ref[...]
    den = d0_ref[...] + d1_ref[...] + ps
    num = p0_ref[...] + p1_ref[...] + ps * ht_ref[...]
    hin = jnp.maximum(num / den + b_ref[...][None, :], 0.0)
    h = jnp.dot(hin, w_ref[...], preferred_element_type=_f32)
    h_ref[...] = h
    _attn_scalars(h, asrc_ref[...], adst_ref[...], ss_ref, sd_ref, c_ref,
                  pself_ref)


def _tc_mid(p0, p1, d0, d1, ps, ht, b, w, asrc, adst):
    return pl.pallas_call(_tc_mid_body, out_shape=_scalar_out)(
        p0, p1, d0, d1, ps, ht, b, w, asrc, adst)


def _tc_headout_body(p0_ref, p1_ref, d0_ref, d1_ref, ps_ref, ht_ref, b_ref,
                     wmt_ref, rs_ref):
    ps = ps_ref[...]
    den = d0_ref[...] + d1_ref[...] + ps
    num = p0_ref[...] + p1_ref[...] + ps * ht_ref[...]
    h = jnp.maximum(num / den + b_ref[...][None, :], 0.0)
    s = jnp.dot(h, wmt_ref[...], preferred_element_type=_f32)
    rows = lax.broadcasted_iota(_i32, (N, SUB), 0)
    cols = lax.broadcasted_iota(_i32, (N, SUB), 1)
    mask = (rows % SUB == cols).astype(_f32)
    ones = jnp.ones((SUB, 1), _f32)
    rs_ref[...] = jnp.dot(s * mask, ones, preferred_element_type=_f32)


def _tc_headout(p0, p1, d0, d1, ps, ht, b, wmt):
    return pl.pallas_call(
        _tc_headout_body,
        out_shape=jax.ShapeDtypeStruct((N, 1), _f32),
    )(p0, p1, d0, d1, ps, ht, b, wmt)


def _tc_fold_body(p_ref, bout_ref, o_ref):
    o_ref[...] = jnp.sum(p_ref[...], axis=1, keepdims=True) + bout_ref[...][None, :]


def _tc_fold(p, bout):
    return pl.pallas_call(
        _tc_fold_body,
        out_shape=jax.ShapeDtypeStruct((N // SUB, 1), _f32),
    )(p, bout)


_mesh = plsc.VectorSubcoreMesh(core_axis_name="c", subcore_axis_name="s")
_sc_params = pltpu.CompilerParams(use_tc_tiling_on_sc=False,
                                  needs_layout_passes=False)


@functools.partial(
    pl.kernel,
    out_type=[
        jax.ShapeDtypeStruct((NPAD,), _f32),
        jax.ShapeDtypeStruct((NPAD,), _f32),
        jax.ShapeDtypeStruct((NPAD, DH), _f32),
        jax.ShapeDtypeStruct((NPAD, DH), _f32),
    ],
    mesh=_mesh,
    compiler_params=_sc_params,
    scratch_types=[
        pltpu.VMEM((R, 128), _i32),
        pltpu.VMEM((R, 128), _i32),
        pltpu.VMEM((R, 128), _f32),
        pltpu.VMEM((N,), _f32),
        pltpu.VMEM((N,), _f32),
        pltpu.VMEM((16,), _f32),
        pltpu.VMEM((64, DH), _f32),
        pltpu.VMEM((NSL,), _f32),
        pltpu.VMEM((128, DH), _f32),
        pltpu.VMEM((128, DH), _f32),
        pltpu.VMEM((128, DH), _f32),
        pltpu.VMEM((128, DH), _f32),
        pltpu.VMEM_SHARED((NPAD,), _f32),
        pltpu.VMEM_SHARED((NPAD, DH), _f32),
        pltpu.SemaphoreType.DMA,
        pltpu.SemaphoreType.DMA,
        pltpu.SemaphoreType.DMA,
        pltpu.SemaphoreType.DMA,
        pltpu.SemaphoreType.DMA,
        pltpu.SemaphoreType.DMA,
    ],
)
def _sc_gat_layer(src_hbm, dst_hbm, ss_hbm, sd_hbm, cv_hbm, h_hbm,
                  d0_hbm, d1_hbm, o0_hbm, o1_hbm,
                  vsrc, vdst, vp, ssv, sdv, cbuf, zbuf, zdbuf,
                  rin0, rin1, rout0, rout1, dsh, osh,
                  semz, semd, semg0, semg1, sems0, sems1):
    c = lax.axis_index("c")
    s = lax.axis_index("s")
    wid = c * NS + s
    base = wid * R
    nrows = jnp.minimum(R, ROWS - base)
    row0 = s * NSL

    def ifire(j, carry):
        pltpu.async_copy(src_hbm.at[pl.ds((base + j) * 128, 128)], vsrc.at[j], semz)
        pltpu.async_copy(dst_hbm.at[pl.ds((base + j) * 128, 128)], vdst.at[j], semz)
        return carry

    lax.fori_loop(0, nrows, ifire, 0)
    pltpu.async_copy(ss_hbm, ssv, semz)
    pltpu.async_copy(sd_hbm, sdv, semz)
    pltpu.sync_copy(cv_hbm, cbuf)

    for r in range(64):
        for t in range(DH // 16):
            zbuf[r, pl.ds(t * 16, 16)] = jnp.zeros((16,), _f32)
    for t in range(NSL // 16):
        zdbuf[pl.ds(t * 16, 16)] = jnp.zeros((16,), _f32)
    for t in range(NSL // 64):
        pltpu.async_copy(zbuf, osh.at[pl.ds(row0 + t * 64, 64)], semg0)
    pltpu.async_copy(zdbuf, dsh.at[pl.ds(s * NSL, NSL)], semg1)

    def idrain(j, carry):
        pltpu.make_async_copy(src_hbm.at[pl.ds(0, 128)], vsrc.at[j], semz).wait()
        pltpu.make_async_copy(src_hbm.at[pl.ds(0, 128)], vdst.at[j], semz).wait()
        return carry

    lax.fori_loop(0, nrows, idrain, 0)
    pltpu.make_async_copy(ss_hbm, ssv, semz).wait()
    pltpu.make_async_copy(sd_hbm, sdv, semz).wait()
    cv = cbuf[...]

    def pbody(j, carry):
        for k in range(8):
            sl = pl.ds(k * 16, 16)
            e = (plsc.load_gather(ssv, [vsrc[j, sl]])
                 + plsc.load_gather(sdv, [vdst[j, sl]]))
            e = jnp.where(e > 0.0, e, 0.2 * e)
            vp[j, sl] = jnp.exp(e - cv)
        return carry

    lax.fori_loop(0, nrows, pbody, 0)

    for t in range(NSL // 64):
        pltpu.make_async_copy(o0_hbm.at[pl.ds(0, 64)], zbuf, semg0).wait()
    pltpu.make_async_copy(d0_hbm.at[pl.ds(0, NSL)], zdbuf, semg1).wait()
    plsc.subcore_barrier()

    def dfire(j, carry):
        pltpu.async_copy(vp.at[j], dsh.at[vdst.at[j]], semd, add=True)
        return carry

    lax.fori_loop(0, nrows, dfire, 0)

    def scale(j, rin, rout):
        for g in range(8):
            av = vp[j, pl.ds(g * 16, 16)]
            for u in range(16):
                i = g * 16 + u
                a = av[u]
                for t in range(DH // 16):
                    sl = pl.ds(t * 16, 16)
                    rout[i, sl] = rin[i, sl] * a

    def fire_gather(j, rin, semg):
        pltpu.async_copy(h_hbm.at[vsrc.at[j]], rin, semg)

    def fire_scatter(j, rout, sems):
        pltpu.async_copy(rout, osh.at[vdst.at[j]], sems, add=True)

    def drain(buf, sem):
        pltpu.make_async_copy(h_hbm.at[pl.ds(0, 128)], buf, sem).wait()

    fire_gather(0, rin0, semg0)
    fire_gather(1, rin1, semg1)

    drain(rin0, semg0)
    scale(0, rin0, rout0)
    fire_scatter(0, rout0, sems0)
    fire_gather(2, rin0, semg0)
    drain(rin1, semg1)
    scale(1, rin1, rout1)
    fire_scatter(1, rout1, sems1)
    fire_gather(3, rin1, semg1)

    def body(g, carry):
        j0 = 2 * g
        drain(rin0, semg0)
        drain(rout0, sems0)
        scale(j0, rin0, rout0)
        fire_scatter(j0, rout0, sems0)
        fire_gather(j0 + 2, rin0, semg0)
        drain(rin1, semg1)
        drain(rout1, sems1)
        scale(j0 + 1, rin1, rout1)
        fire_scatter(j0 + 1, rout1, sems1)

        @pl.when(j0 + 3 < nrows)
        def _():
            fire_gather(j0 + 3, rin1, semg1)

        return carry

    lax.fori_loop(1, (nrows - 1) // 2, body, 0)

    jlast = nrows - 1
    drain(rin0, semg0)
    drain(rout0, sems0)
    scale(jlast, rin0, rout0)
    fire_scatter(jlast, rout0, sems0)
    drain(rout0, sems0)
    drain(rout1, sems1)

    def ddrain(j, carry):
        pltpu.make_async_copy(d0_hbm.at[pl.ds(0, 128)], vp.at[j], semd).wait()
        return carry

    lax.fori_loop(0, nrows, ddrain, 0)

    plsc.subcore_barrier()

    sl1 = pl.ds(s * NSL, NSL)
    slr = pl.ds(row0, NSL)

    @pl.when(c == 0)
    def _():
        pltpu.sync_copy(dsh.at[sl1], d0_hbm.at[sl1])
        pltpu.sync_copy(osh.at[slr], o0_hbm.at[slr])

    @pl.when(c == 1)
    def _():
        pltpu.sync_copy(dsh.at[sl1], d1_hbm.at[sl1])
        pltpu.sync_copy(osh.at[slr], o1_hbm.at[slr])


def kernel(x, edge_index, W1, a_src1, a_dst1, b1, W2, a_src2, a_dst2, b2,
           Wout, bout):
    srcf = edge_index[0].astype(_i32)
    dstf = edge_index[1].astype(_i32)

    asrc1r = jnp.reshape(a_src1, (DH, 1))
    adst1r = jnp.reshape(a_dst1, (DH, 1))
    asrc2r = jnp.reshape(a_src2, (DH, 1))
    adst2r = jnp.reshape(a_dst2, (DH, 1))

    h1, ss1, sd1, c1, ps1 = _tc_head(x, W1, asrc1r, adst1r)
    d0, d1, o0, o1 = _sc_gat_layer(srcf, dstf, jnp.reshape(ss1, (N,)),
                                   jnp.reshape(sd1, (N,)), c1, h1)

    d0r = jnp.reshape(d0[:N], (N, 1))
    d1r = jnp.reshape(d1[:N], (N, 1))
    h2, ss2, sd2, c2, ps2 = _tc_mid(o0[:N], o1[:N], d0r, d1r, ps1, h1, b1,
                                    W2, asrc2r, adst2r)
    e0, e1, q0, q1 = _sc_gat_layer(srcf, dstf, jnp.reshape(ss2, (N,)),
                                   jnp.reshape(sd2, (N,)), c2, h2)

    e0r = jnp.reshape(e0[:N], (N, 1))
    e1r = jnp.reshape(e1[:N], (N, 1))
    wmt = jnp.transpose(jnp.reshape(Wout[:, 0], (SUB, DH)))
    rs = _tc_headout(q0[:N], q1[:N], e0r, e1r, ps2, h2, b2, wmt)
    out = _tc_fold(jnp.reshape(rs, (N // SUB, SUB)), bout)
    return out


# speedup vs baseline: 1.2508x; 1.0389x over previous
import functools

import jax
import jax.numpy as jnp
from jax import lax
from jax.experimental import pallas as pl
from jax.experimental.pallas import tpu as pltpu
from jax.experimental.pallas import tpu_sc as plsc

N = 10000
E0 = 320000
D_IN = 128
DH = 64
SUB = 10

NC = 2
NS = 16
NW = NC * NS
R = 79
ROWS = E0 // 128
NPAD = 10240
NSL = NPAD // NS

_f32 = jnp.float32
_i32 = jnp.int32


def _attn_scalars(h, a2, ss_ref, sd_ref, c_ref, pself_ref):
    ssd = jnp.dot(h, a2, preferred_element_type=_f32)
    ss = ssd[:, 0:1]
    sd = ssd[:, 1:2]
    ss_ref[...] = ss
    sd_ref[...] = sd
    craw = jnp.max(ss) + jnp.max(sd)
    c = jnp.where(craw > 0.0, craw, 0.2 * craw)
    c_ref[...] = jnp.full((16,), c, _f32)
    eself = ss + sd
    eself = jnp.where(eself > 0.0, eself, 0.2 * eself)
    pself_ref[...] = jnp.exp(eself - c)


_scalar_out = [
    jax.ShapeDtypeStruct((N, DH), _f32),
    jax.ShapeDtypeStruct((N, 1), _f32),
    jax.ShapeDtypeStruct((N, 1), _f32),
    jax.ShapeDtypeStruct((16,), _f32),
    jax.ShapeDtypeStruct((N, 1), _f32),
]


def _tc_head_body(x_ref, w_ref, a2_ref,
                  h_ref, ss_ref, sd_ref, c_ref, pself_ref):
    h = jnp.dot(x_ref[...], w_ref[...], preferred_element_type=_f32)
    h_ref[...] = h
    _attn_scalars(h, a2_ref[...], ss_ref, sd_ref, c_ref, pself_ref)


def _tc_head(x, w, a2):
    return pl.pallas_call(_tc_head_body, out_shape=_scalar_out)(x, w, a2)


def _tc_mid_body(p0_ref, p1_ref, d0_ref, d1_ref, ps_ref, ht_ref, b_ref,
                 w_ref, a2_ref,
                 h_ref, ss_ref, sd_ref, c_ref, pself_ref):
    ps = ps_ref[...]
    den = d0_ref[...] + d1_ref[...] + ps
    num = p0_ref[...] + p1_ref[...] + ps * ht_ref[...]
    hin = jnp.maximum(num / den + b_ref[...][None, :], 0.0)
    h = jnp.dot(hin, w_ref[...], preferred_element_type=_f32)
    h_ref[...] = h
    _attn_scalars(h, a2_ref[...], ss_ref, sd_ref, c_ref, pself_ref)


def _tc_mid(p0, p1, d0, d1, ps, ht, b, w, a2):
    return pl.pallas_call(_tc_mid_body, out_shape=_scalar_out)(
        p0, p1, d0, d1, ps, ht, b, w, a2)


def _tc_headout_body(p0_ref, p1_ref, d0_ref, d1_ref, ps_ref, ht_ref, b_ref,
                     wmt_ref, rs_ref):
    ps = ps_ref[...]
    den = d0_ref[...] + d1_ref[...] + ps
    num = p0_ref[...] + p1_ref[...] + ps * ht_ref[...]
    h = jnp.maximum(num / den + b_ref[...][None, :], 0.0)
    s = jnp.dot(h, wmt_ref[...], preferred_element_type=_f32)
    rows = lax.broadcasted_iota(_i32, (N, SUB), 0)
    cols = lax.broadcasted_iota(_i32, (N, SUB), 1)
    mask = (rows % SUB == cols).astype(_f32)
    ones = jnp.ones((SUB, 1), _f32)
    rs_ref[...] = jnp.dot(s * mask, ones, preferred_element_type=_f32)


def _tc_headout(p0, p1, d0, d1, ps, ht, b, wmt):
    return pl.pallas_call(
        _tc_headout_body,
        out_shape=jax.ShapeDtypeStruct((N, 1), _f32),
    )(p0, p1, d0, d1, ps, ht, b, wmt)


def _tc_fold_body(p_ref, bout_ref, o_ref):
    o_ref[...] = jnp.sum(p_ref[...], axis=1, keepdims=True) + bout_ref[...][None, :]


def _tc_fold(p, bout):
    return pl.pallas_call(
        _tc_fold_body,
        out_shape=jax.ShapeDtypeStruct((N // SUB, 1), _f32),
    )(p, bout)


_mesh = plsc.VectorSubcoreMesh(core_axis_name="c", subcore_axis_name="s")
_sc_params = pltpu.CompilerParams(use_tc_tiling_on_sc=False,
                                  needs_layout_passes=False)


@functools.partial(
    pl.kernel,
    out_type=[
        jax.ShapeDtypeStruct((NPAD,), _f32),
        jax.ShapeDtypeStruct((NPAD,), _f32),
        jax.ShapeDtypeStruct((NPAD, DH), _f32),
        jax.ShapeDtypeStruct((NPAD, DH), _f32),
    ],
    mesh=_mesh,
    compiler_params=_sc_params,
    scratch_types=[
        pltpu.VMEM((R, 128), _i32),
        pltpu.VMEM((R, 128), _i32),
        pltpu.VMEM((R, 128), _f32),
        pltpu.VMEM((N,), _f32),
        pltpu.VMEM((N,), _f32),
        pltpu.VMEM((16,), _f32),
        pltpu.VMEM((64, DH), _f32),
        pltpu.VMEM((NSL,), _f32),
        pltpu.VMEM((128, DH), _f32),
        pltpu.VMEM((128, DH), _f32),
        pltpu.VMEM((128, DH), _f32),
        pltpu.VMEM((128, DH), _f32),
        pltpu.VMEM_SHARED((NPAD,), _f32),
        pltpu.VMEM_SHARED((NPAD, DH), _f32),
        pltpu.SemaphoreType.DMA,
        pltpu.SemaphoreType.DMA,
        pltpu.SemaphoreType.DMA,
        pltpu.SemaphoreType.DMA,
        pltpu.SemaphoreType.DMA,
        pltpu.SemaphoreType.DMA,
    ],
)
def _sc_gat_layer(ei_hbm, ss_hbm, sd_hbm, cv_hbm, h_hbm,
                  d0_hbm, d1_hbm, o0_hbm, o1_hbm,
                  vsrc, vdst, vp, ssv, sdv, cbuf, zbuf, zdbuf,
                  rin0, rin1, rout0, rout1, dsh, osh,
                  semz, semd, semg0, semg1, sems0, sems1):
    c = lax.axis_index("c")
    s = lax.axis_index("s")
    wid = c * NS + s
    base = wid * R
    nrows = jnp.minimum(R, ROWS - base)
    row0 = s * NSL

    def ifire(j, carry):
        pltpu.async_copy(ei_hbm.at[0, pl.ds((base + j) * 128, 128)], vsrc.at[j], semz)
        pltpu.async_copy(ei_hbm.at[1, pl.ds((base + j) * 128, 128)], vdst.at[j], semz)
        return carry

    lax.fori_loop(0, nrows, ifire, 0)
    pltpu.async_copy(ss_hbm, ssv, semz)
    pltpu.async_copy(sd_hbm, sdv, semz)
    pltpu.sync_copy(cv_hbm, cbuf)

    for r in range(64):
        for t in range(DH // 16):
            zbuf[r, pl.ds(t * 16, 16)] = jnp.zeros((16,), _f32)
    for t in range(NSL // 16):
        zdbuf[pl.ds(t * 16, 16)] = jnp.zeros((16,), _f32)
    for t in range(NSL // 64):
        pltpu.async_copy(zbuf, osh.at[pl.ds(row0 + t * 64, 64)], semg0)
    pltpu.async_copy(zdbuf, dsh.at[pl.ds(s * NSL, NSL)], semg1)

    def idrain(j, carry):
        pltpu.make_async_copy(ei_hbm.at[0, pl.ds(0, 128)], vsrc.at[j], semz).wait()
        pltpu.make_async_copy(ei_hbm.at[0, pl.ds(0, 128)], vdst.at[j], semz).wait()
        return carry

    lax.fori_loop(0, nrows, idrain, 0)
    pltpu.make_async_copy(ss_hbm, ssv, semz).wait()
    pltpu.make_async_copy(sd_hbm, sdv, semz).wait()
    cv = cbuf[...]

    def pbody(j, carry):
        for k in range(8):
            sl = pl.ds(k * 16, 16)
            e = (plsc.load_gather(ssv, [vsrc[j, sl]])
                 + plsc.load_gather(sdv, [vdst[j, sl]]))
            e = jnp.where(e > 0.0, e, 0.2 * e)
            vp[j, sl] = jnp.exp(e - cv)
        return carry

    lax.fori_loop(0, nrows, pbody, 0)

    for t in range(NSL // 64):
        pltpu.make_async_copy(o0_hbm.at[pl.ds(0, 64)], zbuf, semg0).wait()
    pltpu.make_async_copy(d0_hbm.at[pl.ds(0, NSL)], zdbuf, semg1).wait()
    plsc.subcore_barrier()

    def dfire(j, carry):
        pltpu.async_copy(vp.at[j], dsh.at[vdst.at[j]], semd, add=True)
        return carry

    lax.fori_loop(0, nrows, dfire, 0)

    def scale(j, rin, rout):
        for g in range(8):
            av = vp[j, pl.ds(g * 16, 16)]
            for u in range(16):
                i = g * 16 + u
                a = av[u]
                for t in range(DH // 16):
                    sl = pl.ds(t * 16, 16)
                    rout[i, sl] = rin[i, sl] * a

    def fire_gather(j, rin, semg):
        pltpu.async_copy(h_hbm.at[vsrc.at[j]], rin, semg)

    def fire_scatter(j, rout, sems):
        pltpu.async_copy(rout, osh.at[vdst.at[j]], sems, add=True)

    def drain(buf, sem):
        pltpu.make_async_copy(h_hbm.at[pl.ds(0, 128)], buf, sem).wait()

    fire_gather(0, rin0, semg0)
    fire_gather(1, rin1, semg1)

    drain(rin0, semg0)
    scale(0, rin0, rout0)
    fire_scatter(0, rout0, sems0)
    fire_gather(2, rin0, semg0)
    drain(rin1, semg1)
    scale(1, rin1, rout1)
    fire_scatter(1, rout1, sems1)
    fire_gather(3, rin1, semg1)

    def body(g, carry):
        j0 = 2 * g
        drain(rin0, semg0)
        drain(rout0, sems0)
        scale(j0, rin0, rout0)
        fire_scatter(j0, rout0, sems0)
        fire_gather(j0 + 2, rin0, semg0)
        drain(rin1, semg1)
        drain(rout1, sems1)
        scale(j0 + 1, rin1, rout1)
        fire_scatter(j0 + 1, rout1, sems1)

        @pl.when(j0 + 3 < nrows)
        def _():
            fire_gather(j0 + 3, rin1, semg1)

        return carry

    lax.fori_loop(1, (nrows - 1) // 2, body, 0)

    jlast = nrows - 1
    drain(rin0, semg0)
    drain(rout0, sems0)
    scale(jlast, rin0, rout0)
    fire_scatter(jlast, rout0, sems0)
    drain(rout0, sems0)
    drain(rout1, sems1)

    def ddrain(j, carry):
        pltpu.make_async_copy(d0_hbm.at[pl.ds(0, 128)], vp.at[j], semd).wait()
        return carry

    lax.fori_loop(0, nrows, ddrain, 0)

    plsc.subcore_barrier()

    sl1 = pl.ds(s * NSL, NSL)
    slr = pl.ds(row0, NSL)

    @pl.when(c == 0)
    def _():
        pltpu.sync_copy(dsh.at[sl1], d0_hbm.at[sl1])
        pltpu.sync_copy(osh.at[slr], o0_hbm.at[slr])

    @pl.when(c == 1)
    def _():
        pltpu.sync_copy(dsh.at[sl1], d1_hbm.at[sl1])
        pltpu.sync_copy(osh.at[slr], o1_hbm.at[slr])


def kernel(x, edge_index, W1, a_src1, a_dst1, b1, W2, a_src2, a_dst2, b2,
           Wout, bout):
    ei32 = edge_index.astype(_i32)

    a21 = jnp.stack([a_src1, a_dst1], axis=1)
    a22 = jnp.stack([a_src2, a_dst2], axis=1)

    h1, ss1, sd1, c1, ps1 = _tc_head(x, W1, a21)
    d0, d1, o0, o1 = _sc_gat_layer(ei32, jnp.reshape(ss1, (N,)),
                                   jnp.reshape(sd1, (N,)), c1, h1)

    d0r = jnp.reshape(d0[:N], (N, 1))
    d1r = jnp.reshape(d1[:N], (N, 1))
    h2, ss2, sd2, c2, ps2 = _tc_mid(o0[:N], o1[:N], d0r, d1r, ps1, h1, b1,
                                    W2, a22)
    e0, e1, q0, q1 = _sc_gat_layer(ei32, jnp.reshape(ss2, (N,)),
                                   jnp.reshape(sd2, (N,)), c2, h2)

    e0r = jnp.reshape(e0[:N], (N, 1))
    e1r = jnp.reshape(e1[:N], (N, 1))
    wmt = jnp.transpose(jnp.reshape(Wout[:, 0], (SUB, DH)))
    rs = _tc_headout(q0[:N], q1[:N], e0r, e1r, ps2, h2, b2, wmt)
    out = _tc_fold(jnp.reshape(rs, (N // SUB, SUB)), bout)
    return out
